# Initial kernel scaffold; baseline (speedup 1.0000x reference)
import jax
import jax.numpy as jnp
from jax.experimental import pallas as pl


def kernel(h, edge_index, W1, b1, W2, b2, gamma, beta):
    raise NotImplementedError("write your pallas kernel here")


# speedup vs baseline: 12.2616x; 12.2616x over previous
import functools

import jax
import jax.numpy as jnp
from jax import lax
from jax.experimental import pallas as pl
from jax.experimental.pallas import tpu as pltpu
from jax.experimental.pallas import tpu_sc as plsc

N_NODES = 10000
D_IN = 128
D_HID = 32
D_OUT = 128
E_EDGES = 320000

NC = 2
NS = 16
NW = NC * NS
CHUNK = 128
EPW = -(-E_EDGES // (NW * CHUNK)) * CHUNK
NCHUNK = EPW // CHUNK
E_PAD = NW * EPW
NP = 10240
ZROWS = NP // NS


def _mm_body(h_ref, w_ref, o_ref):
    o_ref[...] = jnp.dot(h_ref[...], w_ref[...],
                         preferred_element_type=jnp.float32)


def _project(h, W1):
    return pl.pallas_call(
        _mm_body,
        out_shape=jax.ShapeDtypeStruct((N_NODES, D_HID), jnp.float32),
    )(h, W1)


_sc_mesh = plsc.VectorSubcoreMesh(core_axis_name="c", subcore_axis_name="s")


@functools.partial(
    pl.kernel,
    out_type=(
        jax.ShapeDtypeStruct((NC, NP, D_HID), jnp.float32),
        jax.ShapeDtypeStruct((NC, NP), jnp.float32),
    ),
    mesh=_sc_mesh,
    compiler_params=pltpu.CompilerParams(use_tc_tiling_on_sc=False),
    scratch_types=[
        pltpu.VMEM((NCHUNK, CHUNK), jnp.int32),
        pltpu.VMEM((NCHUNK, CHUNK), jnp.int32),
        pltpu.VMEM((CHUNK, D_HID), jnp.float32),
        pltpu.VMEM((CHUNK,), jnp.float32),
        pltpu.VMEM((ZROWS, D_HID), jnp.float32),
        pltpu.VMEM((ZROWS,), jnp.float32),
        pltpu.VMEM_SHARED((NP, D_HID), jnp.float32),
        pltpu.VMEM_SHARED((NP,), jnp.float32),
    ],
)
def _sc_segment_sum(p_hbm, src_hbm, dst_hbm, z2_hbm, z1_hbm,
                    s_out, deg_out,
                    srcv, dstv, rows, onesv, zbuf2, zbufd, accs, accd):
    cid = lax.axis_index("c")
    sid = lax.axis_index("s")
    wid = cid * NS + sid

    pltpu.sync_copy(src_hbm.at[wid], srcv)
    pltpu.sync_copy(dst_hbm.at[wid], dstv)

    for t in range(CHUNK // 16):
        onesv[pl.ds(t * 16, 16)] = jnp.ones((16,), jnp.float32)

    pltpu.sync_copy(z2_hbm, zbuf2)
    pltpu.sync_copy(zbuf2, accs.at[pl.ds(sid * ZROWS, ZROWS)])
    pltpu.sync_copy(z1_hbm, zbufd)
    pltpu.sync_copy(zbufd, accd.at[pl.ds(sid * ZROWS, ZROWS)])
    plsc.subcore_barrier()

    def body(j, carry):
        pltpu.sync_copy(p_hbm.at[srcv.at[j]], rows)
        pltpu.sync_copy(rows, accs.at[dstv.at[j]], add=True)
        pltpu.sync_copy(onesv, accd.at[dstv.at[j]], add=True)
        return carry

    lax.fori_loop(0, NCHUNK, body, 0)
    plsc.subcore_barrier()

    @pl.when(sid == 0)
    def _():
        pltpu.sync_copy(accs, s_out.at[cid])
        pltpu.sync_copy(accd, deg_out.at[cid])


def _tail_body(sp_ref, dp_ref, p_ref, b1_ref, w2_ref, b2_ref, g_ref, bt_ref,
               o_ref):
    s = sp_ref[0, :N_NODES, :] + sp_ref[1, :N_NODES, :]
    deg = dp_ref[0, :N_NODES] + dp_ref[1, :N_NODES]
    invd = 1.0 / jnp.maximum(deg, 1.0)
    z1 = jnp.maximum(s * invd[:, None] + p_ref[...] + b1_ref[...], 0.0)
    z = jnp.dot(z1, w2_ref[...], preferred_element_type=jnp.float32)
    z = jnp.maximum(z + b2_ref[...], 0.0)
    mean = jnp.mean(z, axis=0, keepdims=True)
    var = jnp.mean((z - mean) ** 2, axis=0, keepdims=True)
    o_ref[...] = (z - mean) * lax.rsqrt(var + 1e-5) * g_ref[...] + bt_ref[...]


def _tail(s_parts, deg_parts, p, b1, W2, b2, gamma, beta):
    return pl.pallas_call(
        _tail_body,
        out_shape=jax.ShapeDtypeStruct((N_NODES, D_OUT), jnp.float32),
    )(s_parts, deg_parts, p, b1.reshape(1, -1), W2, b2.reshape(1, -1),
      gamma.reshape(1, -1), beta.reshape(1, -1))


def kernel(h, edge_index, W1, b1, W2, b2, gamma, beta):
    p = _project(h, W1)

    src = edge_index[0].astype(jnp.int32)
    dst = edge_index[1].astype(jnp.int32)
    pad = E_PAD - E_EDGES
    src_p = jnp.concatenate(
        [src, jnp.zeros((pad,), jnp.int32)]).reshape(NW, NCHUNK, CHUNK)
    dst_p = jnp.concatenate(
        [dst, jnp.full((pad,), N_NODES, jnp.int32)]).reshape(NW, NCHUNK, CHUNK)

    z2 = jnp.zeros((ZROWS, D_HID), jnp.float32)
    z1 = jnp.zeros((ZROWS,), jnp.float32)
    s_parts, deg_parts = _sc_segment_sum(p, src_p, dst_p, z2, z1)

    return _tail(s_parts, deg_parts, p, b1, W2, b2, gamma, beta)
